# baseline (speedup 1.0000x reference)
import jax
import jax.numpy as jnp
from jax import lax
from jax.experimental import pallas as pl


_B_BLK = 4


def _router_block(x_ref, w_ref, out_ref):
    i = pl.program_id(0)
    s = x_ref.shape[-1] * x_ref.shape[-2]
    pooled = jnp.sum(x_ref[...], axis=(2, 3)) * (1.0 / s)
    out_ref[pl.ds(i * _B_BLK, _B_BLK), :] = lax.dot_general(
        pooled, w_ref[...],
        dimension_numbers=(((1,), (1,)), ((), ())),
        preferred_element_type=jnp.float32,
    )


def kernel(x, W):
    b, c, h, w = x.shape
    e = W.shape[0]
    grid = (b // _B_BLK,)
    return pl.pallas_call(
        _router_block,
        grid=grid,
        in_specs=[
            pl.BlockSpec((_B_BLK, c, h, w), lambda i: (i, 0, 0, 0)),
            pl.BlockSpec((e, c), lambda i: (0, 0)),
        ],
        out_specs=pl.BlockSpec((b, e), lambda i: (0, 0)),
        out_shape=jax.ShapeDtypeStruct((b, e), jnp.float32),
    )(x, W)


# speedup vs baseline: 13.3704x; 13.3704x over previous
import jax
import jax.numpy as jnp
from jax import lax
from jax.experimental import pallas as pl


_B_BLK = 4


def _router_block(x_ref, w_ref, out_ref):
    i = pl.program_id(0)
    s = x_ref.shape[1]
    pooled = jnp.sum(x_ref[...], axis=1) * (1.0 / s)
    out_ref[pl.ds(i * _B_BLK, _B_BLK), :] = lax.dot_general(
        pooled, w_ref[...],
        dimension_numbers=(((1,), (1,)), ((), ())),
        preferred_element_type=jnp.float32,
    )


def kernel(x, W):
    b, c, h, w = x.shape
    e = W.shape[0]
    s = h * w
    xt = jnp.transpose(x, (0, 2, 3, 1)).reshape(b, s, c)
    grid = (b // _B_BLK,)
    return pl.pallas_call(
        _router_block,
        grid=grid,
        in_specs=[
            pl.BlockSpec((_B_BLK, s, c), lambda i: (i, 0, 0)),
            pl.BlockSpec((e, c), lambda i: (0, 0)),
        ],
        out_specs=pl.BlockSpec((b, e), lambda i: (0, 0)),
        out_shape=jax.ShapeDtypeStruct((b, e), jnp.float32),
    )(xt, W)
